# initial kernel scaffold (unmeasured)
import jax
import jax.numpy as jnp
from jax import lax
from jax.experimental import pallas as pl
from jax.experimental.pallas import tpu as pltpu


def kernel(
    x,
):
    def body(*refs):
        pass

    out_shape = jax.ShapeDtypeStruct(..., jnp.float32)
    return pl.pallas_call(body, out_shape=out_shape)(...)


# baseline (device time: 2192671 ns/iter reference)
import jax
import jax.numpy as jnp
from jax import lax
from jax.experimental import pallas as pl
from jax.experimental.pallas import tpu as pltpu


def kernel(x):
    m, n = x.shape
    x = x.astype(jnp.bfloat16)

    def body(x_ref, out_ref, local_sem, send_sem, recv_sem):
        my_x = lax.axis_index("x")
        my_y = lax.axis_index("y")
        my_z = lax.axis_index("z")
        partner = (1 - my_x, my_y, my_z)

        barrier_sem = pltpu.get_barrier_semaphore()
        pl.semaphore_signal(
            barrier_sem, inc=1,
            device_id=partner, device_id_type=pl.DeviceIdType.MESH,
        )
        pl.semaphore_wait(barrier_sem, 1)

        local = pltpu.make_async_copy(
            x_ref, out_ref.at[pl.ds(my_x * m, m), :], local_sem
        )
        local.start()

        rdma = pltpu.make_async_remote_copy(
            src_ref=x_ref,
            dst_ref=out_ref.at[pl.ds(my_x * m, m), :],
            send_sem=send_sem,
            recv_sem=recv_sem,
            device_id=partner,
            device_id_type=pl.DeviceIdType.MESH,
        )
        rdma.start()
        local.wait()
        rdma.wait()

    return pl.pallas_call(
        body,
        out_shape=jax.ShapeDtypeStruct((2 * m, n), jnp.bfloat16),
        in_specs=[pl.BlockSpec(memory_space=pl.ANY)],
        out_specs=pl.BlockSpec(memory_space=pl.ANY),
        scratch_shapes=[
            pltpu.SemaphoreType.DMA,
            pltpu.SemaphoreType.DMA,
            pltpu.SemaphoreType.DMA,
        ],
        compiler_params=pltpu.CompilerParams(collective_id=0),
    )(x)


# device time: 2190563 ns/iter; 1.0010x vs baseline; 1.0010x over previous
import jax
import jax.numpy as jnp
from jax import lax
from jax.experimental import pallas as pl
from jax.experimental.pallas import tpu as pltpu


N_CHUNKS = 16


def kernel(x):
    m, n = x.shape
    x = x.astype(jnp.bfloat16)
    mc = m // N_CHUNKS

    def body(x_ref, out_ref, local_sems, send_sems, recv_sems):
        my_x = lax.axis_index("x")
        my_y = lax.axis_index("y")
        my_z = lax.axis_index("z")
        partner = (1 - my_x, my_y, my_z)

        barrier_sem = pltpu.get_barrier_semaphore()
        pl.semaphore_signal(
            barrier_sem, inc=1,
            device_id=partner, device_id_type=pl.DeviceIdType.MESH,
        )
        pl.semaphore_wait(barrier_sem, 1)

        rdmas = []
        for c in range(N_CHUNKS):
            pltpu.make_async_copy(
                x_ref.at[pl.ds(c * mc, mc), :],
                out_ref.at[pl.ds(my_x * m + c * mc, mc), :],
                local_sems.at[c],
            ).start()
            rdma = pltpu.make_async_remote_copy(
                src_ref=x_ref.at[pl.ds(c * mc, mc), :],
                dst_ref=out_ref.at[pl.ds(my_x * m + c * mc, mc), :],
                send_sem=send_sems.at[c],
                recv_sem=recv_sems.at[c],
                device_id=partner,
                device_id_type=pl.DeviceIdType.MESH,
            )
            rdma.start()
            rdmas.append(rdma)

        for c in range(N_CHUNKS):
            pltpu.make_async_copy(
                x_ref.at[pl.ds(c * mc, mc), :],
                out_ref.at[pl.ds(my_x * m + c * mc, mc), :],
                local_sems.at[c],
            ).wait()
            rdmas[c].wait()

    return pl.pallas_call(
        body,
        out_shape=jax.ShapeDtypeStruct((2 * m, n), jnp.bfloat16),
        in_specs=[pl.BlockSpec(memory_space=pl.ANY)],
        out_specs=pl.BlockSpec(memory_space=pl.ANY),
        scratch_shapes=[
            pltpu.SemaphoreType.DMA((N_CHUNKS,)),
            pltpu.SemaphoreType.DMA((N_CHUNKS,)),
            pltpu.SemaphoreType.DMA((N_CHUNKS,)),
        ],
        compiler_params=pltpu.CompilerParams(collective_id=0),
    )(x)


# device time: 874828 ns/iter; 2.5064x vs baseline; 2.5040x over previous
import jax
import jax.numpy as jnp
from jax import lax
from jax.experimental import pallas as pl
from jax.experimental.pallas import tpu as pltpu

N_CHUNKS = 16


def kernel(x):
    m, n = x.shape
    x = x.astype(jnp.bfloat16)
    mc = m // N_CHUNKS

    def body(x_ref, out_ref, stage_buf, load_sems, store_sems,
             send_sems, recv_sems):
        my_x = lax.axis_index("x")
        my_y = lax.axis_index("y")
        my_z = lax.axis_index("z")
        partner = (1 - my_x, my_y, my_z)

        barrier_sem = pltpu.get_barrier_semaphore()
        pl.semaphore_signal(
            barrier_sem, inc=1,
            device_id=partner, device_id_type=pl.DeviceIdType.MESH,
        )
        pl.semaphore_wait(barrier_sem, 1)

        rdmas = []
        for c in range(N_CHUNKS):
            rdma = pltpu.make_async_remote_copy(
                src_ref=x_ref.at[pl.ds(c * mc, mc), :],
                dst_ref=out_ref.at[pl.ds(my_x * m + c * mc, mc), :],
                send_sem=send_sems.at[c],
                recv_sem=recv_sems.at[c],
                device_id=partner,
                device_id_type=pl.DeviceIdType.MESH,
            )
            rdma.start()
            rdmas.append(rdma)

        def load(c):
            return pltpu.make_async_copy(
                x_ref.at[pl.ds(c * mc, mc), :],
                stage_buf.at[c % 2],
                load_sems.at[c % 2],
            )

        def store(c):
            return pltpu.make_async_copy(
                stage_buf.at[c % 2],
                out_ref.at[pl.ds(my_x * m + c * mc, mc), :],
                store_sems.at[c % 2],
            )

        load(0).start()
        for c in range(N_CHUNKS):
            load(c).wait()
            store(c).start()
            if c + 1 < N_CHUNKS:
                if c >= 1:
                    store(c - 1).wait()
                load(c + 1).start()
        store(N_CHUNKS - 2).wait()
        store(N_CHUNKS - 1).wait()

        for c in range(N_CHUNKS):
            rdmas[c].wait()

    return pl.pallas_call(
        body,
        out_shape=jax.ShapeDtypeStruct((2 * m, n), jnp.bfloat16),
        in_specs=[pl.BlockSpec(memory_space=pl.ANY)],
        out_specs=pl.BlockSpec(memory_space=pl.ANY),
        scratch_shapes=[
            pltpu.VMEM((2, m // N_CHUNKS, n), jnp.bfloat16),
            pltpu.SemaphoreType.DMA((2,)),
            pltpu.SemaphoreType.DMA((2,)),
            pltpu.SemaphoreType.DMA((N_CHUNKS,)),
            pltpu.SemaphoreType.DMA((N_CHUNKS,)),
        ],
        compiler_params=pltpu.CompilerParams(collective_id=0),
    )(x)


# device time: 812666 ns/iter; 2.6981x vs baseline; 1.0765x over previous
import jax
import jax.numpy as jnp
from jax import lax
from jax.experimental import pallas as pl
from jax.experimental.pallas import tpu as pltpu

N_CHUNKS = 16


def kernel(x):
    m, n = x.shape
    mc = m // N_CHUNKS

    def body(x_ref, out_ref, f32_buf, bf16_buf, load_sems, store_sems,
             send_sems, recv_sems):
        my_x = lax.axis_index("x")
        my_y = lax.axis_index("y")
        my_z = lax.axis_index("z")
        partner = (1 - my_x, my_y, my_z)

        barrier_sem = pltpu.get_barrier_semaphore()
        pl.semaphore_signal(
            barrier_sem, inc=1,
            device_id=partner, device_id_type=pl.DeviceIdType.MESH,
        )
        pl.semaphore_wait(barrier_sem, 1)

        def load(c):
            return pltpu.make_async_copy(
                x_ref.at[pl.ds(c * mc, mc), :],
                f32_buf.at[c % 2],
                load_sems.at[c % 2],
            )

        def store(c):
            return pltpu.make_async_copy(
                bf16_buf.at[c % 4],
                out_ref.at[pl.ds(my_x * m + c * mc, mc), :],
                store_sems.at[c % 4],
            )

        load(0).start()
        rdmas = []
        for c in range(N_CHUNKS):
            load(c).wait()
            if c + 1 < N_CHUNKS:
                load(c + 1).start()
            if c >= 4:
                rdmas[c - 4].wait()
                store(c - 4).wait()
            bf16_buf[c % 4, :, :] = f32_buf[c % 2, :, :].astype(
                jnp.bfloat16
            )
            rdma = pltpu.make_async_remote_copy(
                src_ref=bf16_buf.at[c % 4],
                dst_ref=out_ref.at[pl.ds(my_x * m + c * mc, mc), :],
                send_sem=send_sems.at[c],
                recv_sem=recv_sems.at[c],
                device_id=partner,
                device_id_type=pl.DeviceIdType.MESH,
            )
            rdma.start()
            rdmas.append(rdma)
            store(c).start()
        for c in range(N_CHUNKS - 4, N_CHUNKS):
            rdmas[c].wait()
            store(c).wait()

    return pl.pallas_call(
        body,
        out_shape=jax.ShapeDtypeStruct((2 * m, n), jnp.bfloat16),
        in_specs=[pl.BlockSpec(memory_space=pl.ANY)],
        out_specs=pl.BlockSpec(memory_space=pl.ANY),
        scratch_shapes=[
            pltpu.VMEM((2, m // N_CHUNKS, n), jnp.float32),
            pltpu.VMEM((4, m // N_CHUNKS, n), jnp.bfloat16),
            pltpu.SemaphoreType.DMA((2,)),
            pltpu.SemaphoreType.DMA((4,)),
            pltpu.SemaphoreType.DMA((N_CHUNKS,)),
            pltpu.SemaphoreType.DMA((N_CHUNKS,)),
        ],
        compiler_params=pltpu.CompilerParams(collective_id=0),
    )(x)
